# baseline (speedup 1.0000x reference)
import jax
import jax.numpy as jnp
from jax import lax
from jax.experimental import pallas as pl
from jax.experimental.pallas import tpu as pltpu

N_TOK = 65536
C = 1024
D = 64
BT = 2048
NB = N_TOK // BT


def _codebook_kernel(mp_ref, eps_ref, w2_ref, p2_ref, phit_ref):
    mp = mp_ref[...]
    eps = eps_ref[...]
    phi = jnp.dot(mp, eps, preferred_element_type=jnp.float32)
    w2_ref[...] = -2.0 * phi
    p2_ref[...] = jnp.sum(phi * phi, axis=1, keepdims=True)
    phit_ref[...] = phi.T


def _vq_kernel(xt_ref, w2_ref, p2_ref, phit_ref, out_ref):
    xt = xt_ref[...]
    xp2 = jnp.dot(w2_ref[...], xt, preferred_element_type=jnp.float32)
    d2t = xp2 + p2_ref[...]
    idx = jnp.argmin(d2t, axis=0, keepdims=True).astype(jnp.int32)
    ids = lax.broadcasted_iota(jnp.int32, d2t.shape, 0)
    onehot = jnp.where(ids == idx, 1.0, 0.0)
    out_ref[...] = jnp.dot(phit_ref[...], onehot,
                           preferred_element_type=jnp.float32,
                           precision=jax.lax.Precision.HIGHEST)


def kernel(X, matrix_parents, epsilon):
    w2, p2, phit = pl.pallas_call(
        _codebook_kernel,
        out_shape=[
            jax.ShapeDtypeStruct((C, D), jnp.float32),
            jax.ShapeDtypeStruct((C, 1), jnp.float32),
            jax.ShapeDtypeStruct((D, C), jnp.float32),
        ],
    )(matrix_parents, epsilon)

    out_t = pl.pallas_call(
        _vq_kernel,
        grid=(NB,),
        in_specs=[
            pl.BlockSpec((D, BT), lambda i: (0, i)),
            pl.BlockSpec((C, D), lambda i: (0, 0)),
            pl.BlockSpec((C, 1), lambda i: (0, 0)),
            pl.BlockSpec((D, C), lambda i: (0, 0)),
        ],
        out_specs=pl.BlockSpec((D, BT), lambda i: (0, i)),
        out_shape=jax.ShapeDtypeStruct((D, N_TOK), jnp.float32),
        compiler_params=pltpu.CompilerParams(
            dimension_semantics=("arbitrary",)),
    )(X.T, w2, p2, phit)
    return out_t.T
